# Initial kernel scaffold; baseline (speedup 1.0000x reference)
import functools

import jax
import jax.numpy as jnp
from jax import lax
from jax.experimental import pallas as pl
from jax.experimental.pallas import tpu as pltpu
from jax.experimental.pallas import tpu_sc as plsc

B, L, E = 4096, 200, 32
NC, NS = 2, 16
NW = NC * NS
RPW = B // NW
HALF = L // 2


def _sc_embed_pool(inputs, table):
  mesh = plsc.VectorSubcoreMesh(core_axis_name="c", subcore_axis_name="s")

  @functools.partial(
      pl.kernel,
      out_type=(
          jax.ShapeDtypeStruct((B, L, E), jnp.float32),
          jax.ShapeDtypeStruct((B, E), jnp.float32),
      ),
      mesh=mesh,
      scratch_types=[
          pltpu.VMEM((L,), jnp.int32),
          pltpu.VMEM((L, E), jnp.float32),
          pltpu.VMEM((RPW, E), jnp.float32),
          pltpu.SemaphoreType.DMA,
      ],
  )
  def k(inputs_hbm, table_hbm, emb_hbm, pool_hbm, idx_v, rows_v, pool_v, sem):
    wid = lax.axis_index("s") * NC + lax.axis_index("c")
    base = wid * RPW

    def row_body(i, carry):
      b = base + i
      pltpu.sync_copy(inputs_hbm.at[b], idx_v)
      cp0 = pltpu.async_copy(
          table_hbm.at[idx_v.at[pl.ds(0, HALF)]], rows_v.at[pl.ds(0, HALF)],
          sem)
      cp1 = pltpu.async_copy(
          table_hbm.at[idx_v.at[pl.ds(HALF, HALF)]],
          rows_v.at[pl.ds(HALF, HALF)], sem)
      cp0.wait()
      cp1.wait()
      out_cp = pltpu.async_copy(rows_v, emb_hbm.at[b], sem)

      def red(j, accs):
        a0, a1, a2, a3 = accs
        a0 = a0 + rows_v[2 * j, pl.ds(0, 16)]
        a1 = a1 + rows_v[2 * j, pl.ds(16, 16)]
        a2 = a2 + rows_v[2 * j + 1, pl.ds(0, 16)]
        a3 = a3 + rows_v[2 * j + 1, pl.ds(16, 16)]
        return (a0, a1, a2, a3)

      zero = jnp.zeros((16,), jnp.float32)
      a0, a1, a2, a3 = lax.fori_loop(
          0, L // 2, red, (zero, zero, zero, zero), unroll=4)
      scale = jnp.float32(1.0 / L)
      pool_v[i, pl.ds(0, 16)] = (a0 + a2) * scale
      pool_v[i, pl.ds(16, 16)] = (a1 + a3) * scale
      out_cp.wait()
      return carry

    lax.fori_loop(0, RPW, row_body, 0)
    pltpu.sync_copy(pool_v, pool_hbm.at[pl.ds(base, RPW)])

  return k(inputs, table)


def _tc_dense(pool, fc_W, fc_b, cls_W, cls_b):
  def body(p_ref, w1_ref, b1_ref, w2_ref, b2_ref, fc_ref, out_ref):
    fc = jnp.maximum(
        jnp.dot(p_ref[...], w1_ref[...], preferred_element_type=jnp.float32)
        + b1_ref[...], 0.0)
    fc_ref[...] = fc
    out_ref[...] = jnp.dot(
        fc, w2_ref[...], preferred_element_type=jnp.float32) + b2_ref[...]

  return pl.pallas_call(
      body,
      out_shape=(
          jax.ShapeDtypeStruct((B, 64), jnp.float32),
          jax.ShapeDtypeStruct((B, 2), jnp.float32),
      ),
  )(pool, fc_W, fc_b.reshape(1, 64), cls_W, cls_b.reshape(1, 2))


def kernel(inputs, table, fc_W, fc_b, cls_W, cls_b):
  emb, pool = _sc_embed_pool(inputs, table)
  fc_x, logits = _tc_dense(pool, fc_W, fc_b, cls_W, cls_b)
  return (logits, emb, pool, fc_x)


# speedup vs baseline: 1.3177x; 1.3177x over previous
import functools

import jax
import jax.numpy as jnp
from jax import lax
from jax.experimental import pallas as pl
from jax.experimental.pallas import tpu as pltpu
from jax.experimental.pallas import tpu_sc as plsc

B, L, E = 4096, 200, 32
NC, NS = 2, 16
NW = NC * NS
RPW = B // NW
SPLIT = 128
REST = L - SPLIT


def _sc_embed_pool(inputs, table):
  mesh = plsc.VectorSubcoreMesh(core_axis_name="c", subcore_axis_name="s")

  @functools.partial(
      pl.kernel,
      out_type=(
          jax.ShapeDtypeStruct((B, L, E), jnp.float32),
          jax.ShapeDtypeStruct((B, E), jnp.float32),
      ),
      mesh=mesh,
      compiler_params=pltpu.CompilerParams(use_tc_tiling_on_sc=False),
      scratch_types=[
          pltpu.VMEM((L,), jnp.int32),
          pltpu.VMEM((L, E), jnp.float32),
          pltpu.VMEM((RPW, E), jnp.float32),
          pltpu.SemaphoreType.DMA,
      ],
  )
  def k(inputs_hbm, table_hbm, emb_hbm, pool_hbm, idx_v, rows_v, pool_v, sem):
    wid = lax.axis_index("s") * NC + lax.axis_index("c")
    base = wid * RPW

    def row_body(i, carry):
      b = base + i
      pltpu.sync_copy(inputs_hbm.at[b], idx_v)
      cp0 = pltpu.async_copy(
          table_hbm.at[idx_v.at[pl.ds(0, SPLIT)]], rows_v.at[pl.ds(0, SPLIT)],
          sem)
      cp1 = pltpu.async_copy(
          table_hbm.at[idx_v.at[pl.ds(SPLIT, REST)]],
          rows_v.at[pl.ds(SPLIT, REST)], sem)
      cp0.wait()
      cp1.wait()
      out_cp = pltpu.async_copy(rows_v, emb_hbm.at[b], sem)

      def red(j, accs):
        a0, a1, a2, a3 = accs
        a0 = a0 + rows_v[2 * j, pl.ds(0, 16)]
        a1 = a1 + rows_v[2 * j, pl.ds(16, 16)]
        a2 = a2 + rows_v[2 * j + 1, pl.ds(0, 16)]
        a3 = a3 + rows_v[2 * j + 1, pl.ds(16, 16)]
        return (a0, a1, a2, a3)

      zero = jnp.zeros((16,), jnp.float32)
      a0, a1, a2, a3 = lax.fori_loop(
          0, L // 2, red, (zero, zero, zero, zero), unroll=4)
      scale = jnp.float32(1.0 / L)
      pool_v[i, pl.ds(0, 16)] = (a0 + a2) * scale
      pool_v[i, pl.ds(16, 16)] = (a1 + a3) * scale
      out_cp.wait()
      return carry

    lax.fori_loop(0, RPW, row_body, 0)
    pltpu.sync_copy(pool_v, pool_hbm.at[pl.ds(base, RPW)])

  return k(inputs, table)


def _tc_dense(pool, fc_W, fc_b, cls_W, cls_b):
  def body(p_ref, w1_ref, b1_ref, w2_ref, b2_ref, fc_ref, out_ref):
    fc = jnp.maximum(
        jnp.dot(p_ref[...], w1_ref[...], preferred_element_type=jnp.float32)
        + b1_ref[...], 0.0)
    fc_ref[...] = fc
    out_ref[...] = jnp.dot(
        fc, w2_ref[...], preferred_element_type=jnp.float32) + b2_ref[...]

  return pl.pallas_call(
      body,
      out_shape=(
          jax.ShapeDtypeStruct((B, 64), jnp.float32),
          jax.ShapeDtypeStruct((B, 2), jnp.float32),
      ),
  )(pool, fc_W, fc_b.reshape(1, 64), cls_W, cls_b.reshape(1, 2))


def kernel(inputs, table, fc_W, fc_b, cls_W, cls_b):
  emb, pool = _sc_embed_pool(inputs, table)
  fc_x, logits = _tc_dense(pool, fc_W, fc_b, cls_W, cls_b)
  return (logits, emb, pool, fc_x)


# speedup vs baseline: 1.5273x; 1.1591x over previous
import functools

import jax
import jax.numpy as jnp
from jax import lax
from jax.experimental import pallas as pl
from jax.experimental.pallas import tpu as pltpu
from jax.experimental.pallas import tpu_sc as plsc

B, L, E = 4096, 200, 32
NC, NS = 2, 16
NW = NC * NS
RPW = B // NW
SPLIT = 128
REST = L - SPLIT
NBUF = 4


def _sc_embed_pool(inputs, table):
  mesh = plsc.VectorSubcoreMesh(core_axis_name="c", subcore_axis_name="s")

  @functools.partial(
      pl.kernel,
      out_type=(
          jax.ShapeDtypeStruct((B, L, E), jnp.float32),
          jax.ShapeDtypeStruct((B, E), jnp.float32),
      ),
      mesh=mesh,
      compiler_params=pltpu.CompilerParams(use_tc_tiling_on_sc=False),
      scratch_types=[
          pltpu.VMEM((RPW, L), jnp.int32),
          pltpu.VMEM((NBUF, L, E), jnp.float32),
          pltpu.VMEM((RPW, E), jnp.float32),
          pltpu.SemaphoreType.DMA((NBUF,)),
          pltpu.SemaphoreType.DMA((NBUF,)),
      ],
  )
  def k(inputs_hbm, table_hbm, emb_hbm, pool_hbm, idx_v, bufs, pool_v, gsem,
        osem):
    wid = lax.axis_index("s") * NC + lax.axis_index("c")
    base = wid * RPW

    pltpu.sync_copy(inputs_hbm.at[pl.ds(base, RPW)], idx_v)

    def gather_row(r, b, start):
      cp0 = pltpu.make_async_copy(
          table_hbm.at[idx_v.at[r, pl.ds(0, SPLIT)]],
          bufs.at[b, pl.ds(0, SPLIT)], gsem.at[b])
      cp1 = pltpu.make_async_copy(
          table_hbm.at[idx_v.at[r, pl.ds(SPLIT, REST)]],
          bufs.at[b, pl.ds(SPLIT, REST)], gsem.at[b])
      if start:
        cp0.start()
        cp1.start()
      else:
        cp0.wait()
        cp1.wait()

    for b in range(NBUF):
      gather_row(b, b, True)

    def group(g, carry):
      for b in range(NBUF):
        r = g * NBUF + b
        gather_row(r, b, False)
        out_cp = pltpu.async_copy(bufs.at[b], emb_hbm.at[base + r],
                                  osem.at[b])

        def red(j, accs):
          a0, a1, a2, a3 = accs
          a0 = a0 + bufs[b, 2 * j, pl.ds(0, 16)]
          a1 = a1 + bufs[b, 2 * j, pl.ds(16, 16)]
          a2 = a2 + bufs[b, 2 * j + 1, pl.ds(0, 16)]
          a3 = a3 + bufs[b, 2 * j + 1, pl.ds(16, 16)]
          return (a0, a1, a2, a3)

        zero = jnp.zeros((16,), jnp.float32)
        a0, a1, a2, a3 = lax.fori_loop(
            0, L // 2, red, (zero, zero, zero, zero), unroll=4)
        scale = jnp.float32(1.0 / L)
        pool_v[r, pl.ds(0, 16)] = (a0 + a2) * scale
        pool_v[r, pl.ds(16, 16)] = (a1 + a3) * scale

        out_cp.wait()

        @pl.when(r < RPW - NBUF)
        def _():
          gather_row(r + NBUF, b, True)

      return carry

    lax.fori_loop(0, RPW // NBUF, group, 0)
    pltpu.sync_copy(pool_v, pool_hbm.at[pl.ds(base, RPW)])

  return k(inputs, table)


def _tc_dense(pool, fc_W, fc_b, cls_W, cls_b):
  def body(p_ref, w1_ref, b1_ref, w2_ref, b2_ref, fc_ref, out_ref):
    fc = jnp.maximum(
        jnp.dot(p_ref[...], w1_ref[...], preferred_element_type=jnp.float32)
        + b1_ref[...], 0.0)
    fc_ref[...] = fc
    out_ref[...] = jnp.dot(
        fc, w2_ref[...], preferred_element_type=jnp.float32) + b2_ref[...]

  return pl.pallas_call(
      body,
      out_shape=(
          jax.ShapeDtypeStruct((B, 64), jnp.float32),
          jax.ShapeDtypeStruct((B, 2), jnp.float32),
      ),
  )(pool, fc_W, fc_b.reshape(1, 64), cls_W, cls_b.reshape(1, 2))


def kernel(inputs, table, fc_W, fc_b, cls_W, cls_b):
  emb, pool = _sc_embed_pool(inputs, table)
  fc_x, logits = _tc_dense(pool, fc_W, fc_b, cls_W, cls_b)
  return (logits, emb, pool, fc_x)
